# baseline (speedup 1.0000x reference)
import functools

import jax
import jax.numpy as jnp
from jax import lax
from jax.experimental import pallas as pl
from jax.experimental.pallas import tpu as pltpu
from jax.experimental.pallas import tpu_sc as plsc

NEG_SLOPE = 0.01
LANES = 16
XCHUNK = 1000


def _seg_sum_edges(edge_index, en, nn, e_total, n_nodes, num_workers):
    epw = e_total // num_workers
    NWIN = 5
    ew = epw // NWIN
    nch = ew // LANES
    ALIGN = 512
    win = -(-(ew + ALIGN) // ALIGN) * ALIGN
    max_start = e_total - win
    mesh = plsc.VectorSubcoreMesh(core_axis_name="c", subcore_axis_name="s")

    @functools.partial(
        pl.kernel,
        mesh=mesh,
        out_type=jax.ShapeDtypeStruct((num_workers, n_nodes), jnp.float32),
        compiler_params=pltpu.CompilerParams(needs_layout_passes=False),
        scratch_types=[
            pltpu.VMEM((2, win), jnp.int32),
            pltpu.VMEM((2, win), jnp.int32),
            pltpu.VMEM((ew,), jnp.float32),
            pltpu.VMEM((ew,), jnp.float32),
            pltpu.VMEM((n_nodes,), jnp.float32),
            pltpu.VMEM((n_nodes,), jnp.float32),
            pltpu.SemaphoreType.DMA,
            pltpu.SemaphoreType.DMA,
            pltpu.SemaphoreType.DMA,
        ],
    )
    def k(ei_hbm, en_hbm, nn_hbm, out_hbm,
          ei_a, ei_b, en_a, en_b, nn_v, s_v, sem_a, sem_b, sem_n):
        c = lax.axis_index("c")
        s = lax.axis_index("s")
        wid = s * 2 + c
        base = wid * epw
        ei_bufs = (ei_a, ei_b)
        en_bufs = (en_a, en_b)
        sems = (sem_a, sem_b)

        starts = []
        offs = []
        for j in range(NWIN):
            wbase = base + j * ew
            st = jnp.minimum((wbase // ALIGN) * ALIGN, max_start)
            starts.append(st)
            offs.append(wbase - st)

        def fire(j):
            cpe = pltpu.async_copy(
                ei_hbm.at[:, pl.ds(starts[j], win)], ei_bufs[j % 2], sems[j % 2])
            cpn = pltpu.async_copy(
                en_hbm.at[pl.ds(base + j * ew, ew)], en_bufs[j % 2], sems[j % 2])
            return cpe, cpn

        cps = [fire(0)]
        cp3 = pltpu.async_copy(nn_hbm, nn_v, sem_n)

        @plsc.parallel_loop(0, n_nodes // LANES, unroll=8)
        def _(i):
            s_v[pl.ds(i * LANES, LANES)] = jnp.zeros((LANES,), jnp.float32)

        cp3.wait()
        for j in range(NWIN):
            if j + 1 < NWIN:
                cps.append(fire(j + 1))
            for cp in cps[j]:
                cp.wait()
            ei_v = ei_bufs[j % 2]
            en_v = en_bufs[j % 2]
            off = offs[j]

            @plsc.parallel_loop(0, nch, unroll=8)
            def _(i):
                sl = pl.ds(i * LANES, LANES)
                wsl = pl.ds(off + i * LANES, LANES)
                w = plsc.load_gather(nn_v, [ei_v[1, wsl]]) * en_v[sl]
                plsc.addupdate_scatter(s_v, [ei_v[0, wsl]], w)

        pltpu.sync_copy(s_v, out_hbm.at[wid])

    return k(edge_index, en, nn)


def _dense_readout(s_part, nn_row, x, w, b, n_nodes):
    def body(sp_ref, nn_ref, x_ref, w_ref, b_ref, o_ref):
        s2 = jnp.sum(sp_ref[...], axis=0, keepdims=True) * nn_ref[...]
        r = jnp.dot(s2, x_ref[...], preferred_element_type=jnp.float32)
        z = lax.dot_general(r * (1.0 / n_nodes), w_ref[...],
                            (((1,), (1,)), ((), ())),
                            preferred_element_type=jnp.float32) + b_ref[...][None, :]
        o_ref[...] = jnp.where(z >= 0, z, NEG_SLOPE * z)

    return pl.pallas_call(
        body,
        out_shape=jax.ShapeDtypeStruct((1, x.shape[1]), jnp.float32),
    )(s_part, nn_row, x, w, b)


def kernel(x, edge_index, node_norm, edge_norm, W, b):
    n_nodes = x.shape[0]
    e_total = edge_index.shape[1]
    s_part = _seg_sum_edges(edge_index, edge_norm, node_norm,
                            e_total, n_nodes, 32)
    return _dense_readout(s_part, node_norm, x, W, b, n_nodes)


# speedup vs baseline: 1.0451x; 1.0451x over previous
import functools

import jax
import jax.numpy as jnp
from jax import lax
from jax.experimental import pallas as pl
from jax.experimental.pallas import tpu as pltpu
from jax.experimental.pallas import tpu_sc as plsc

NEG_SLOPE = 0.01
LANES = 16
XCHUNK = 1000


def _seg_sum_edges(edge_index, en, nn, e_total, n_nodes, num_workers):
    epw = e_total // num_workers
    nch = epw // LANES
    ALIGN = 512
    win = -(-(epw + ALIGN) // ALIGN) * ALIGN
    max_start = e_total - win
    mesh = plsc.VectorSubcoreMesh(core_axis_name="c", subcore_axis_name="s")

    @functools.partial(
        pl.kernel,
        mesh=mesh,
        out_type=jax.ShapeDtypeStruct((num_workers, n_nodes), jnp.float32),
        compiler_params=pltpu.CompilerParams(needs_layout_passes=False),
        scratch_types=[
            pltpu.VMEM((2, win), jnp.int32),
            pltpu.VMEM((epw,), jnp.float32),
            pltpu.VMEM((n_nodes,), jnp.float32),
            pltpu.VMEM((n_nodes,), jnp.float32),
            pltpu.SemaphoreType.DMA,
        ],
    )
    def k(ei_hbm, en_hbm, nn_hbm, out_hbm, ei_v, en_v, nn_v, s_v, sem):
        c = lax.axis_index("c")
        s = lax.axis_index("s")
        wid = s * 2 + c
        base = wid * epw
        start = jnp.minimum((base // ALIGN) * ALIGN, max_start)
        off = base - start
        cp0 = pltpu.async_copy(ei_hbm.at[:, pl.ds(start, win)], ei_v, sem)
        cp2 = pltpu.async_copy(en_hbm.at[pl.ds(base, epw)], en_v, sem)
        cp3 = pltpu.async_copy(nn_hbm, nn_v, sem)

        @plsc.parallel_loop(0, n_nodes // LANES, unroll=8)
        def _(i):
            s_v[pl.ds(i * LANES, LANES)] = jnp.zeros((LANES,), jnp.float32)

        cp0.wait()
        cp2.wait()
        cp3.wait()

        @plsc.parallel_loop(0, nch, unroll=16)
        def _(i):
            sl = pl.ds(i * LANES, LANES)
            wsl = pl.ds(off + i * LANES, LANES)
            w = plsc.load_gather(nn_v, [ei_v[1, wsl]]) * en_v[sl]
            plsc.addupdate_scatter(s_v, [ei_v[0, wsl]], w)

        pltpu.sync_copy(s_v, out_hbm.at[wid])

    return k(edge_index, en, nn)


def _dense_readout(s_part, nn_row, x, w, b, n_nodes):
    def body(sp_ref, nn_ref, x_ref, w_ref, b_ref, o_ref):
        s2 = jnp.sum(sp_ref[...], axis=0, keepdims=True) * nn_ref[...]
        r = jnp.dot(s2, x_ref[...], preferred_element_type=jnp.float32)
        z = lax.dot_general(r * (1.0 / n_nodes), w_ref[...],
                            (((1,), (1,)), ((), ())),
                            preferred_element_type=jnp.float32) + b_ref[...][None, :]
        o_ref[...] = jnp.where(z >= 0, z, NEG_SLOPE * z)

    return pl.pallas_call(
        body,
        out_shape=jax.ShapeDtypeStruct((1, x.shape[1]), jnp.float32),
    )(s_part, nn_row, x, w, b)


def kernel(x, edge_index, node_norm, edge_norm, W, b):
    n_nodes = x.shape[0]
    e_total = edge_index.shape[1]
    s_part = _seg_sum_edges(edge_index, edge_norm, node_norm,
                            e_total, n_nodes, 32)
    return _dense_readout(s_part, node_norm, x, W, b, n_nodes)


# speedup vs baseline: 1.0482x; 1.0029x over previous
import functools

import jax
import jax.numpy as jnp
from jax import lax
from jax.experimental import pallas as pl
from jax.experimental.pallas import tpu as pltpu
from jax.experimental.pallas import tpu_sc as plsc

NEG_SLOPE = 0.01
LANES = 16


def _seg_sum_edges(edge_index, en, nn, e_total, n_nodes, num_workers):
    epw = e_total // num_workers
    nch = epw // LANES
    ALIGN = 512
    win = -(-(epw + ALIGN) // ALIGN) * ALIGN
    max_start = e_total - win
    mesh = plsc.VectorSubcoreMesh(core_axis_name="c", subcore_axis_name="s")

    @functools.partial(
        pl.kernel,
        mesh=mesh,
        out_type=jax.ShapeDtypeStruct((num_workers, n_nodes), jnp.float32),
        compiler_params=pltpu.CompilerParams(needs_layout_passes=False),
        scratch_types=[
            pltpu.VMEM((2, win), jnp.int32),
            pltpu.VMEM((epw,), jnp.float32),
            pltpu.VMEM((n_nodes,), jnp.float32),
            pltpu.VMEM((n_nodes,), jnp.float32),
            pltpu.SemaphoreType.DMA,
        ],
    )
    def k(ei_hbm, en_hbm, nn_hbm, out_hbm, ei_v, en_v, nn_v, s_v, sem):
        c = lax.axis_index("c")
        s = lax.axis_index("s")
        wid = s * 2 + c
        base = wid * epw
        start = jnp.minimum((base // ALIGN) * ALIGN, max_start)
        off = base - start
        cp0 = pltpu.async_copy(ei_hbm.at[:, pl.ds(start, win)], ei_v, sem)
        cp2 = pltpu.async_copy(en_hbm.at[pl.ds(base, epw)], en_v, sem)
        cp3 = pltpu.async_copy(nn_hbm, nn_v, sem)

        @plsc.parallel_loop(0, n_nodes // LANES, unroll=8)
        def _(i):
            s_v[pl.ds(i * LANES, LANES)] = jnp.zeros((LANES,), jnp.float32)

        cp0.wait()
        cp2.wait()
        cp3.wait()

        @plsc.parallel_loop(0, nch, unroll=16)
        def _(i):
            sl = pl.ds(i * LANES, LANES)
            wsl = pl.ds(off + i * LANES, LANES)
            w = plsc.load_gather(nn_v, [ei_v[1, wsl]]) * en_v[sl]
            plsc.addupdate_scatter(s_v, [ei_v[0, wsl]], w)

        pltpu.sync_copy(s_v, out_hbm.at[wid])

    return k(edge_index, en, nn)


def _dense_readout(s_part, nn_row, x, w, b, n_nodes):
    def body(sp_ref, nn_ref, x_ref, w_ref, b_ref, o_ref):
        s2 = jnp.sum(sp_ref[...], axis=0, keepdims=True) * nn_ref[...]
        r = jnp.dot(s2, x_ref[...], preferred_element_type=jnp.float32)
        z = lax.dot_general(r * (1.0 / n_nodes), w_ref[...],
                            (((1,), (1,)), ((), ())),
                            preferred_element_type=jnp.float32) + b_ref[...][None, :]
        o_ref[...] = jnp.where(z >= 0, z, NEG_SLOPE * z)

    return pl.pallas_call(
        body,
        out_shape=jax.ShapeDtypeStruct((1, x.shape[1]), jnp.float32),
    )(s_part, nn_row, x, w, b)


def kernel(x, edge_index, node_norm, edge_norm, W, b):
    n_nodes = x.shape[0]
    e_total = edge_index.shape[1]
    s_part = _seg_sum_edges(edge_index, edge_norm, node_norm,
                            e_total, n_nodes, 32)
    return _dense_readout(s_part, node_norm, x, W, b, n_nodes)
